# baseline (speedup 1.0000x reference)
import numpy as np
import jax
import jax.numpy as jnp
from jax.experimental import pallas as pl
from jax.experimental.pallas import tpu as pltpu

_D = 384
_K = 8
_SIGMA = _D ** 0.5
_EPS = 1e-6


def _filter_matrix():
    pos = np.arange(-_D // 2 + 1, _D // 2 + 1, dtype=np.float64)
    g = np.exp(-0.5 * (pos / _SIGMA) ** 2)
    g = g / g.max()
    w = np.fft.ifftshift(g)
    eye = np.eye(_D, dtype=np.float64)
    m = np.fft.ifft(np.fft.fft(eye, axis=-1) * w, axis=-1).real
    return jnp.asarray(m - eye, dtype=jnp.float32)


def _body(x_ref, xp_ref, m_ref, pooled_ref, votes_ref, sel_ref, sbuf):
    i = pl.program_id(0)
    n = x_ref.shape[1]
    d = x_ref.shape[2]
    slot = jax.lax.rem(i, 2)
    pslot = jax.lax.rem(i + 1, 2)

    scores = sbuf[pslot]
    xp = xp_ref[0]

    x = x_ref[0]
    resid = jax.lax.dot(
        x, m_ref[...],
        precision=jax.lax.Precision.HIGHEST,
        preferred_element_type=jnp.float32,
    )
    sbuf[slot] = x / jnp.maximum(jnp.abs(resid), _EPS)

    iota = jax.lax.broadcasted_iota(jnp.int32, (n, d), 0)
    neg = jnp.float32(-jnp.inf)
    idx = None
    for k in range(_K):
        if k > 0:
            scores = jnp.where(iota == idx, neg, scores)
        s, ix = scores, iota
        h = n // 2
        while h >= 8:
            ge = s[:h] >= s[h:]
            s, ix = jnp.where(ge, s[:h], s[h:]), jnp.where(ge, ix[:h], ix[h:])
            h //= 2
        m = jnp.max(s, axis=0, keepdims=True)
        idx = jnp.min(jnp.where(s == m, ix, n), axis=0, keepdims=True)
        sel_ref[0, k, :] = idx[0]
    chosen = (scores == neg) | (iota == idx)
    pooled_ref[0, 0, :] = jnp.sum(jnp.where(chosen, xp, 0.0), axis=0) * (1.0 / _K)
    votes_ref[0] = jnp.sum(chosen.astype(jnp.int32), axis=1, keepdims=True)


def kernel(patch_tokens):
    b, n, d = patch_tokens.shape
    m = _filter_matrix()
    pooled, votes, sel = pl.pallas_call(
        _body,
        grid=(b + 1,),
        in_specs=[
            pl.BlockSpec((1, n, d), lambda i: (jnp.minimum(i, b - 1), 0, 0)),
            pl.BlockSpec((1, n, d), lambda i: (jnp.maximum(i - 1, 0), 0, 0)),
            pl.BlockSpec((d, d), lambda i: (0, 0)),
        ],
        out_specs=[
            pl.BlockSpec((1, 1, d), lambda i: (jnp.maximum(i - 1, 0), 0, 0)),
            pl.BlockSpec((1, n, 1), lambda i: (jnp.maximum(i - 1, 0), 0, 0)),
            pl.BlockSpec((1, _K, d), lambda i: (jnp.maximum(i - 1, 0), 0, 0)),
        ],
        out_shape=[
            jax.ShapeDtypeStruct((b, 1, d), jnp.float32),
            jax.ShapeDtypeStruct((b, n, 1), jnp.int32),
            jax.ShapeDtypeStruct((b, _K, d), jnp.int32),
        ],
        scratch_shapes=[
            pltpu.VMEM((2, n, d), jnp.float32),
        ],
        compiler_params=pltpu.CompilerParams(
            dimension_semantics=("arbitrary",),
        ),
    )(patch_tokens, patch_tokens, m)
    return pooled.reshape(b, d), votes.reshape(b, n), sel


# speedup vs baseline: 1.2241x; 1.2241x over previous
import numpy as np
import jax
import jax.numpy as jnp
from jax.experimental import pallas as pl
from jax.experimental.pallas import tpu as pltpu

_D = 384
_K = 8
_SIGMA = _D ** 0.5
_EPS = 1e-6


def _filter_matrix():
    pos = np.arange(-_D // 2 + 1, _D // 2 + 1, dtype=np.float64)
    g = np.exp(-0.5 * (pos / _SIGMA) ** 2)
    g = g / g.max()
    w = np.fft.ifftshift(g)
    eye = np.eye(_D, dtype=np.float64)
    m = np.fft.ifft(np.fft.fft(eye, axis=-1) * w, axis=-1).real
    return jnp.asarray(m - eye, dtype=jnp.float32)


def _body(x_ref, xp_ref, m_ref, pooled_ref, votes_ref, sel_ref, sbuf):
    i = pl.program_id(0)
    n = x_ref.shape[1]
    d = x_ref.shape[2]
    slot = jax.lax.rem(i, 2)
    pslot = jax.lax.rem(i + 1, 2)

    scores = sbuf[pslot]
    xp = xp_ref[0]

    x = x_ref[0]
    resid = jax.lax.dot(
        x, m_ref[...],
        precision=jax.lax.Precision.HIGHEST,
        preferred_element_type=jnp.float32,
    )
    sbuf[slot] = x / jnp.maximum(jnp.abs(resid), _EPS)

    iota = jax.lax.broadcasted_iota(jnp.int32, (n, d), 0)
    neg = jnp.float32(-jnp.inf)
    idx = None
    half = n // 2
    for k in range(_K):
        if k > 0:
            scores = jnp.where(iota == idx, neg, scores)
        s = scores
        h = half
        while h >= 8:
            s = jnp.maximum(s[:h], s[h:])
            h //= 2
        m = jnp.max(s, axis=0, keepdims=True)
        c = jnp.minimum(
            jnp.where(scores[:half] == m, iota[:half], n),
            jnp.where(scores[half:] == m, iota[half:], n),
        )
        h = half // 2
        while h >= 8:
            c = jnp.minimum(c[:h], c[h:])
            h //= 2
        idx = jnp.min(c, axis=0, keepdims=True)
        sel_ref[0, k, :] = idx[0]
    chosen = (scores == neg) | (iota == idx)
    pooled_ref[0, 0, :] = jnp.sum(jnp.where(chosen, xp, 0.0), axis=0) * (1.0 / _K)
    votes_ref[0] = jnp.sum(chosen.astype(jnp.int32), axis=1, keepdims=True)


def kernel(patch_tokens):
    b, n, d = patch_tokens.shape
    m = _filter_matrix()
    pooled, votes, sel = pl.pallas_call(
        _body,
        grid=(b + 1,),
        in_specs=[
            pl.BlockSpec((1, n, d), lambda i: (jnp.minimum(i, b - 1), 0, 0)),
            pl.BlockSpec((1, n, d), lambda i: (jnp.maximum(i - 1, 0), 0, 0)),
            pl.BlockSpec((d, d), lambda i: (0, 0)),
        ],
        out_specs=[
            pl.BlockSpec((1, 1, d), lambda i: (jnp.maximum(i - 1, 0), 0, 0)),
            pl.BlockSpec((1, n, 1), lambda i: (jnp.maximum(i - 1, 0), 0, 0)),
            pl.BlockSpec((1, _K, d), lambda i: (jnp.maximum(i - 1, 0), 0, 0)),
        ],
        out_shape=[
            jax.ShapeDtypeStruct((b, 1, d), jnp.float32),
            jax.ShapeDtypeStruct((b, n, 1), jnp.int32),
            jax.ShapeDtypeStruct((b, _K, d), jnp.int32),
        ],
        scratch_shapes=[
            pltpu.VMEM((2, n, d), jnp.float32),
        ],
        compiler_params=pltpu.CompilerParams(
            dimension_semantics=("arbitrary",),
        ),
    )(patch_tokens, patch_tokens, m)
    return pooled.reshape(b, d), votes.reshape(b, n), sel
